# baseline (speedup 1.0000x reference)
import functools

import jax
import jax.numpy as jnp
from jax import lax
from jax.experimental import pallas as pl
from jax.experimental.pallas import tpu as pltpu
from jax.experimental.pallas import tpu_sc as plsc

N = 10000
E = 320000
D = 128
AVG_D_LOG = 3.4965

NCORE = 2
NSUB = 16
NWORK = NCORE * NSUB
LANE = 16
FV = D // LANE

C = 160
NCH = 64
NP = C * NCH
CPT = NCH // NWORK
B = 1280
NB = E // B
G = 128

R = 1024
NRB = NP // R

_BIG = 3.0e38


def _sc_agg_body(x_hbm, src_hbm, dst_hbm,
                 s_hbm, q_hbm, mx_hbm, mn_hbm, dg_hbm,
                 acc_s, acc_q, acc_x, acc_n, acc_d,
                 srcb, dstb, csrc, cdst, rows, sem):
    wid = lax.axis_index("s") * NCORE + lax.axis_index("c")
    zeros = jnp.zeros((LANE,), jnp.float32)
    ones = jnp.ones((LANE,), jnp.float32)
    neg = jnp.full((LANE,), -_BIG, jnp.float32)
    pos = jnp.full((LANE,), _BIG, jnp.float32)
    izeros = jnp.zeros((LANE,), jnp.int32)

    for ch in range(CPT):
        chunk = wid * CPT + ch
        base = chunk * C

        @pl.loop(0, C)
        def _init(r):
            for f in range(FV):
                sl = pl.ds(f * LANE, LANE)
                acc_s[r, sl] = zeros
                acc_q[r, sl] = zeros
                acc_x[r, sl] = neg
                acc_n[r, sl] = pos
            acc_d[r, pl.ds(0, LANE)] = zeros

        @pl.loop(0, NB)
        def _block(b):
            pltpu.sync_copy(dst_hbm.at[pl.ds(b * B, B)], dstb)
            pltpu.sync_copy(src_hbm.at[pl.ds(b * B, B)], srcb)

            def scan_step(i, cnt):
                dv = dstb[pl.ds(i * LANE, LANE)]
                sv = srcb[pl.ds(i * LANE, LANE)]
                rel = dv - base
                m = (rel >= 0) & (rel < C)
                lane = lax.iota(jnp.int32, LANE)
                key = jnp.where(m, lane, lane + LANE)
                cdst[pl.ds(cnt, LANE)] = plsc.sort_key_val(key, rel)[1]
                csrc[pl.ds(cnt, LANE)] = plsc.sort_key_val(key, sv)[1]
                pc = plsc.all_reduce_population_count(m)
                return cnt + pc[0]

            cnt = lax.fori_loop(0, B // LANE, scan_step, 0)

            for t in range(G // LANE):
                csrc[pl.ds(cnt + t * LANE, LANE)] = izeros

            nsb = (cnt + G - 1) // G

            @pl.loop(0, nsb)
            def _subbatch(sb):
                off = sb * G
                pltpu.async_copy(x_hbm.at[csrc.at[pl.ds(off, G)]], rows, sem).wait()
                hi = jnp.minimum(cnt - off, G) * 0

                @pl.loop(0, hi)
                def _edge(j):
                    dl = cdst[pl.ds(off + j, LANE)][0]
                    for f in range(FV):
                        sl = pl.ds(f * LANE, LANE)
                        mv = rows[j, sl]
                        plsc.addupdate(acc_s.at[dl, sl], mv)
                        plsc.addupdate(acc_q.at[dl, sl], mv * mv)
                        acc_x[dl, sl] = jnp.maximum(acc_x[dl, sl], mv)
                        acc_n[dl, sl] = jnp.minimum(acc_n[dl, sl], mv)
                    plsc.addupdate(acc_d.at[dl, pl.ds(0, LANE)], ones)

        pltpu.sync_copy(acc_s, s_hbm.at[pl.ds(base, C)])
        pltpu.sync_copy(acc_q, q_hbm.at[pl.ds(base, C)])
        pltpu.sync_copy(acc_x, mx_hbm.at[pl.ds(base, C)])
        pltpu.sync_copy(acc_n, mn_hbm.at[pl.ds(base, C)])
        half = C // 2
        for p in range(2):
            @pl.loop(0, half)
            def _expand(r):
                v = acc_d[p * half + r, pl.ds(0, LANE)]
                for f in range(FV):
                    rows[r, pl.ds(f * LANE, LANE)] = v
            pltpu.sync_copy(rows.at[pl.ds(0, half)],
                            dg_hbm.at[pl.ds(base + p * half, half)])


_stat = jax.ShapeDtypeStruct((NP, D), jnp.float32)


@functools.cache
def _sc_agg_call():
    return functools.partial(
        pl.kernel,
        out_type=[_stat] * 5,
        mesh=plsc.VectorSubcoreMesh(core_axis_name="c", subcore_axis_name="s",
                                    num_cores=NCORE, num_subcores=NSUB),
        compiler_params=pltpu.CompilerParams(needs_layout_passes=False),
        scratch_types=[
            pltpu.VMEM((C, D), jnp.float32),
            pltpu.VMEM((C, D), jnp.float32),
            pltpu.VMEM((C, D), jnp.float32),
            pltpu.VMEM((C, D), jnp.float32),
            pltpu.VMEM((C, LANE), jnp.float32),
            pltpu.VMEM((B,), jnp.int32),
            pltpu.VMEM((B,), jnp.int32),
            pltpu.VMEM((B + G,), jnp.int32),
            pltpu.VMEM((B + G,), jnp.int32),
            pltpu.VMEM((G, D), jnp.float32),
            pltpu.SemaphoreType.DMA,
        ],
    )(_sc_agg_body)


def _scalers(sb, qb, xb, nb, db):
    degc = jnp.maximum(db, 1.0)
    mean = sb / degc
    var = jnp.maximum(qb / degc - mean * mean, 0.0)
    std = jnp.sqrt(var + 1e-5)
    has = db > 0.0
    mxm = jnp.where(has, xb, 0.0)
    mnm = jnp.where(has, nb, 0.0)
    logd = jnp.log(db + 1.0)
    amp = logd / AVG_D_LOG
    att = jnp.where(has, AVG_D_LOG / jnp.maximum(logd, 1e-6), 1.0)
    agg = jnp.concatenate([mean, mxm, mnm, std], axis=1)
    ampt = jnp.concatenate([amp] * 4, axis=1)
    attt = jnp.concatenate([att] * 4, axis=1)
    return agg, ampt, attt


def _mm_body(s_ref, q_ref, x_ref, n_ref, d_ref, w_ref, b_ref,
             out_ref, cs_ref, cq_ref):
    i = pl.program_id(0)
    agg, ampt, attt = _scalers(s_ref[...], q_ref[...], x_ref[...], n_ref[...],
                               d_ref[...])
    w = w_ref[...]
    out = jnp.dot(agg, w[0:4 * D], preferred_element_type=jnp.float32)
    out += jnp.dot(agg * ampt, w[4 * D:8 * D], preferred_element_type=jnp.float32)
    out += jnp.dot(agg * attt, w[8 * D:12 * D], preferred_element_type=jnp.float32)
    out += b_ref[...]
    out_ref[...] = out

    @pl.when(i == 0)
    def _():
        cs_ref[...] = jnp.zeros_like(cs_ref)
        cq_ref[...] = jnp.zeros_like(cq_ref)

    rid = lax.broadcasted_iota(jnp.int32, (R, D), 0) + i * R
    om = jnp.where(rid < N, out, 0.0)
    cs_ref[...] += jnp.sum(om, axis=0, keepdims=True)
    cq_ref[...] += jnp.sum(om * om, axis=0, keepdims=True)


def _mm_last_body(s_ref, q_ref, x_ref, n_ref, d_ref, w_ref, b_ref, h_ref,
                  out_ref):
    agg, ampt, attt = _scalers(s_ref[...], q_ref[...], x_ref[...], n_ref[...],
                               d_ref[...])
    w = w_ref[...]
    out = jnp.dot(agg, w[0:4 * D], preferred_element_type=jnp.float32)
    out += jnp.dot(agg * ampt, w[4 * D:8 * D], preferred_element_type=jnp.float32)
    out += jnp.dot(agg * attt, w[8 * D:12 * D], preferred_element_type=jnp.float32)
    out_ref[...] = out + b_ref[...] + h_ref[...]


def _bn_body(out_ref, h_ref, cs_ref, cq_ref, g_ref, b_ref, new_ref):
    mu = cs_ref[...] / N
    var = cq_ref[...] / N - mu * mu
    inv = lax.rsqrt(var + 1e-5)
    y = (out_ref[...] - mu) * inv * g_ref[...] + b_ref[...]
    new_ref[...] = h_ref[...] + jnp.maximum(y, 0.0)


_row_spec = pl.BlockSpec((R, D), lambda i: (i, 0))
_full_w = pl.BlockSpec((12 * D, D), lambda i: (0, 0))
_vec_spec = pl.BlockSpec((1, D), lambda i: (0, 0))

_mm_call = pl.pallas_call(
    _mm_body,
    grid=(NRB,),
    in_specs=[_row_spec] * 5 + [_full_w, _vec_spec],
    out_specs=[_row_spec, _vec_spec, _vec_spec],
    out_shape=[
        jax.ShapeDtypeStruct((NP, D), jnp.float32),
        jax.ShapeDtypeStruct((1, D), jnp.float32),
        jax.ShapeDtypeStruct((1, D), jnp.float32),
    ],
)

_mm_last_call = pl.pallas_call(
    _mm_last_body,
    grid=(NRB,),
    in_specs=[_row_spec] * 5 + [_full_w, _vec_spec, _row_spec],
    out_specs=_row_spec,
    out_shape=jax.ShapeDtypeStruct((NP, D), jnp.float32),
)

_bn_call = pl.pallas_call(
    _bn_body,
    grid=(NRB,),
    in_specs=[_row_spec, _row_spec, _vec_spec, _vec_spec, _vec_spec, _vec_spec],
    out_specs=_row_spec,
    out_shape=jax.ShapeDtypeStruct((NP, D), jnp.float32),
)


def kernel(h, e, W0, b0, W1, b1, W2, b2, W3, b3,
           gamma0, beta0, gamma1, beta1, gamma2, beta2, edge_index):
    del e
    src = edge_index[0]
    dst = edge_index[1]
    Ws = [W0, W1, W2, W3]
    bs = [b.reshape(1, D) for b in (b0, b1, b2, b3)]
    gammas = [g.reshape(1, D) for g in (gamma0, gamma1, gamma2)]
    betas = [b.reshape(1, D) for b in (beta0, beta1, beta2)]

    x = jnp.pad(h, ((0, NP - N), (0, 0)))
    for i in range(4):
        s, q, mx, mn, dg = _sc_agg_call()(x, src, dst)
        if i < 3:
            out, cs, cq = _mm_call(s, q, mx, mn, dg, Ws[i], bs[i])
            x = _bn_call(out, x, cs, cq, gammas[i], betas[i])
        else:
            x = _mm_last_call(s, q, mx, mn, dg, Ws[i], bs[i], x)
    return x[:N]


# speedup vs baseline: 30.5877x; 30.5877x over previous
import functools

import jax
import jax.numpy as jnp
from jax import lax
from jax.experimental import pallas as pl
from jax.experimental.pallas import tpu as pltpu
from jax.experimental.pallas import tpu_sc as plsc

N = 10000
E = 320000
D = 128
AVG_D_LOG = 3.4965

NCORE = 2
NSUB = 16
NWORK = NCORE * NSUB
LANE = 16
FV = D // LANE

C = 160
NCH = 64
NP = C * NCH
CPT = NCH // NWORK
B = 1280
NB = E // B
G = 128

R = 1024
NRB = NP // R

_BIG = 3.0e38


def _sc_agg_body(x_hbm, src_hbm, dst_hbm,
                 s_hbm, q_hbm, mx_hbm, mn_hbm, dg_hbm,
                 acc_s, acc_q, acc_x, acc_n, acc_d,
                 srcb, dstb, csrc, cdst, rows, sem):
    wid = lax.axis_index("s") * NCORE + lax.axis_index("c")
    zeros = jnp.zeros((LANE,), jnp.float32)
    ones = jnp.ones((LANE,), jnp.float32)
    neg = jnp.full((LANE,), -_BIG, jnp.float32)
    pos = jnp.full((LANE,), _BIG, jnp.float32)
    lane = lax.iota(jnp.int32, LANE)

    def do_batch(off, count):
        pltpu.async_copy(x_hbm.at[csrc.at[pl.ds(off, G)]], rows, sem).wait()

        @pl.loop(0, count)
        def _edge(j):
            dl = cdst[pl.ds(off + j, LANE)][0]
            for f in range(FV):
                sl = pl.ds(f * LANE, LANE)
                mv = rows[j, sl]
                plsc.addupdate(acc_s.at[dl, sl], mv)
                plsc.addupdate(acc_q.at[dl, sl], mv * mv)
                acc_x[dl, sl] = jnp.maximum(acc_x[dl, sl], mv)
                acc_n[dl, sl] = jnp.minimum(acc_n[dl, sl], mv)
            plsc.addupdate(acc_d.at[dl, pl.ds(0, LANE)], ones)

    for ch in range(CPT):
        chunk = wid * CPT + ch
        base = chunk * C

        @pl.loop(0, C)
        def _init(r):
            for f in range(FV):
                sl = pl.ds(f * LANE, LANE)
                acc_s[r, sl] = zeros
                acc_q[r, sl] = zeros
                acc_x[r, sl] = neg
                acc_n[r, sl] = pos
            acc_d[r, pl.ds(0, LANE)] = zeros

        def block_step(b, cnt):
            pltpu.sync_copy(dst_hbm.at[pl.ds(b * B, B)], dstb)
            pltpu.sync_copy(src_hbm.at[pl.ds(b * B, B)], srcb)

            def scan_step(i, cnt):
                dv = dstb[pl.ds(i * LANE, LANE)]
                sv = srcb[pl.ds(i * LANE, LANE)]
                rel = dv - base
                m = (rel >= 0) & (rel < C)
                key = jnp.where(m, lane, lane + LANE)
                cdst[pl.ds(cnt, LANE)] = plsc.sort_key_val(key, rel)[1]
                csrc[pl.ds(cnt, LANE)] = plsc.sort_key_val(key, sv)[1]
                pc = plsc.all_reduce_population_count(m)
                return cnt + pc[0]

            cnt = lax.fori_loop(0, B // LANE, scan_step, cnt)

            nfull = cnt // G

            @pl.loop(0, nfull)
            def _full(sb):
                do_batch(sb * G, G)

            @pl.loop(0, G // LANE)
            def _move(t):
                o = nfull * G + t * LANE
                cdst[pl.ds(t * LANE, LANE)] = cdst[pl.ds(o, LANE)]
                csrc[pl.ds(t * LANE, LANE)] = csrc[pl.ds(o, LANE)]

            return cnt - nfull * G

        cnt = lax.fori_loop(0, NB, block_step, 0)

        for t in range(G // LANE):
            csrc[pl.ds(cnt + t * LANE, LANE)] = lane + (t * LANE)

        @pl.when(cnt > 0)
        def _tail():
            do_batch(0, cnt)

        pltpu.sync_copy(acc_s, s_hbm.at[pl.ds(base, C)])
        pltpu.sync_copy(acc_q, q_hbm.at[pl.ds(base, C)])
        pltpu.sync_copy(acc_x, mx_hbm.at[pl.ds(base, C)])
        pltpu.sync_copy(acc_n, mn_hbm.at[pl.ds(base, C)])
        half = C // 2
        for p in range(2):
            @pl.loop(0, half)
            def _expand(r):
                v = acc_d[p * half + r, pl.ds(0, LANE)]
                for f in range(FV):
                    rows[r, pl.ds(f * LANE, LANE)] = v
            pltpu.sync_copy(rows.at[pl.ds(0, half)],
                            dg_hbm.at[pl.ds(base + p * half, half)])


_stat = jax.ShapeDtypeStruct((NP, D), jnp.float32)


@functools.cache
def _sc_agg_call():
    return functools.partial(
        pl.kernel,
        out_type=[_stat] * 5,
        mesh=plsc.VectorSubcoreMesh(core_axis_name="c", subcore_axis_name="s",
                                    num_cores=NCORE, num_subcores=NSUB),
        compiler_params=pltpu.CompilerParams(needs_layout_passes=False),
        scratch_types=[
            pltpu.VMEM((C, D), jnp.float32),
            pltpu.VMEM((C, D), jnp.float32),
            pltpu.VMEM((C, D), jnp.float32),
            pltpu.VMEM((C, D), jnp.float32),
            pltpu.VMEM((C, LANE), jnp.float32),
            pltpu.VMEM((B,), jnp.int32),
            pltpu.VMEM((B,), jnp.int32),
            pltpu.VMEM((B + G,), jnp.int32),
            pltpu.VMEM((B + G,), jnp.int32),
            pltpu.VMEM((G, D), jnp.float32),
            pltpu.SemaphoreType.DMA,
        ],
    )(_sc_agg_body)


def _scalers(sb, qb, xb, nb, db):
    degc = jnp.maximum(db, 1.0)
    mean = sb / degc
    var = jnp.maximum(qb / degc - mean * mean, 0.0)
    std = jnp.sqrt(var + 1e-5)
    has = db > 0.0
    mxm = jnp.where(has, xb, 0.0)
    mnm = jnp.where(has, nb, 0.0)
    logd = jnp.log(db + 1.0)
    amp = logd / AVG_D_LOG
    att = jnp.where(has, AVG_D_LOG / jnp.maximum(logd, 1e-6), 1.0)
    agg = jnp.concatenate([mean, mxm, mnm, std], axis=1)
    ampt = jnp.concatenate([amp] * 4, axis=1)
    attt = jnp.concatenate([att] * 4, axis=1)
    return agg, ampt, attt


def _mm_body(s_ref, q_ref, x_ref, n_ref, d_ref, w_ref, b_ref,
             out_ref, cs_ref, cq_ref):
    i = pl.program_id(0)
    agg, ampt, attt = _scalers(s_ref[...], q_ref[...], x_ref[...], n_ref[...],
                               d_ref[...])
    w = w_ref[...]
    out = jnp.dot(agg, w[0:4 * D], preferred_element_type=jnp.float32)
    out += jnp.dot(agg * ampt, w[4 * D:8 * D], preferred_element_type=jnp.float32)
    out += jnp.dot(agg * attt, w[8 * D:12 * D], preferred_element_type=jnp.float32)
    out += b_ref[...]
    out_ref[...] = out

    @pl.when(i == 0)
    def _():
        cs_ref[...] = jnp.zeros_like(cs_ref)
        cq_ref[...] = jnp.zeros_like(cq_ref)

    rid = lax.broadcasted_iota(jnp.int32, (R, D), 0) + i * R
    om = jnp.where(rid < N, out, 0.0)
    cs_ref[...] += jnp.sum(om, axis=0, keepdims=True)
    cq_ref[...] += jnp.sum(om * om, axis=0, keepdims=True)


def _mm_last_body(s_ref, q_ref, x_ref, n_ref, d_ref, w_ref, b_ref, h_ref,
                  out_ref):
    agg, ampt, attt = _scalers(s_ref[...], q_ref[...], x_ref[...], n_ref[...],
                               d_ref[...])
    w = w_ref[...]
    out = jnp.dot(agg, w[0:4 * D], preferred_element_type=jnp.float32)
    out += jnp.dot(agg * ampt, w[4 * D:8 * D], preferred_element_type=jnp.float32)
    out += jnp.dot(agg * attt, w[8 * D:12 * D], preferred_element_type=jnp.float32)
    out_ref[...] = out + b_ref[...] + h_ref[...]


def _bn_body(out_ref, h_ref, cs_ref, cq_ref, g_ref, b_ref, new_ref):
    mu = cs_ref[...] / N
    var = cq_ref[...] / N - mu * mu
    inv = lax.rsqrt(var + 1e-5)
    y = (out_ref[...] - mu) * inv * g_ref[...] + b_ref[...]
    new_ref[...] = h_ref[...] + jnp.maximum(y, 0.0)


_row_spec = pl.BlockSpec((R, D), lambda i: (i, 0))
_full_w = pl.BlockSpec((12 * D, D), lambda i: (0, 0))
_vec_spec = pl.BlockSpec((1, D), lambda i: (0, 0))

_mm_call = pl.pallas_call(
    _mm_body,
    grid=(NRB,),
    in_specs=[_row_spec] * 5 + [_full_w, _vec_spec],
    out_specs=[_row_spec, _vec_spec, _vec_spec],
    out_shape=[
        jax.ShapeDtypeStruct((NP, D), jnp.float32),
        jax.ShapeDtypeStruct((1, D), jnp.float32),
        jax.ShapeDtypeStruct((1, D), jnp.float32),
    ],
)

_mm_last_call = pl.pallas_call(
    _mm_last_body,
    grid=(NRB,),
    in_specs=[_row_spec] * 5 + [_full_w, _vec_spec, _row_spec],
    out_specs=_row_spec,
    out_shape=jax.ShapeDtypeStruct((NP, D), jnp.float32),
)

_bn_call = pl.pallas_call(
    _bn_body,
    grid=(NRB,),
    in_specs=[_row_spec, _row_spec, _vec_spec, _vec_spec, _vec_spec, _vec_spec],
    out_specs=_row_spec,
    out_shape=jax.ShapeDtypeStruct((NP, D), jnp.float32),
)


def kernel(h, e, W0, b0, W1, b1, W2, b2, W3, b3,
           gamma0, beta0, gamma1, beta1, gamma2, beta2, edge_index):
    del e
    src = edge_index[0]
    dst = edge_index[1]
    Ws = [W0, W1, W2, W3]
    bs = [b.reshape(1, D) for b in (b0, b1, b2, b3)]
    gammas = [g.reshape(1, D) for g in (gamma0, gamma1, gamma2)]
    betas = [b.reshape(1, D) for b in (beta0, beta1, beta2)]

    x = jnp.pad(h, ((0, NP - N), (0, 0)))
    for i in range(4):
        s, q, mx, mn, dg = _sc_agg_call()(x, src, dst)
        if i < 3:
            out, cs, cq = _mm_call(s, q, mx, mn, dg, Ws[i], bs[i])
            x = _bn_call(out, x, cs, cq, gammas[i], betas[i])
        else:
            x = _mm_last_call(s, q, mx, mn, dg, Ws[i], bs[i], x)
    return x[:N]


# speedup vs baseline: 58.4894x; 1.9122x over previous
import functools

import jax
import jax.numpy as jnp
from jax import lax
from jax.experimental import pallas as pl
from jax.experimental.pallas import tpu as pltpu
from jax.experimental.pallas import tpu_sc as plsc

N = 10000
E = 320000
D = 128
AVG_D_LOG = 3.4965

NCORE = 2
NSUB = 16
NWORK = NCORE * NSUB
LANE = 16
FV = D // LANE

C = 160
NCH = 64
NP = C * NCH
CPT = NCH // NWORK
B = 1280
NB = E // B
G = 64

R = 1024
NRB = NP // R

_BIG = 3.0e38

_BKT = NWORK * CPT * E


def _bucket_body(src_hbm, dst_hbm, bsrc_hbm, bdst_hbm, counts_hbm,
                 srcb, dstb, cs0, cd0, cs1, cd1, cv):
    wid = lax.axis_index("s") * NCORE + lax.axis_index("c")
    base = wid * (CPT * C)
    seg0 = wid * (CPT * E)
    seg1 = seg0 + E
    lane = lax.iota(jnp.int32, LANE)

    def flush(cs, cd, cnt, fl, seg):
        nf = cnt // G

        @pl.loop(0, nf)
        def _f(sb):
            o = pl.multiple_of(seg + fl + sb * G, G)
            pltpu.sync_copy(cs.at[pl.ds(sb * G, G)],
                            bsrc_hbm.at[pl.ds(o, G)])
            pltpu.sync_copy(cd.at[pl.ds(sb * G, G)],
                            bdst_hbm.at[pl.ds(o, G)])

        @pl.loop(0, G // LANE)
        def _m(t):
            o = nf * G + t * LANE
            cs[pl.ds(t * LANE, LANE)] = cs[pl.ds(o, LANE)]
            cd[pl.ds(t * LANE, LANE)] = cd[pl.ds(o, LANE)]

        return cnt - nf * G, fl + nf * G

    def block_step(b, carry):
        cnt0, cnt1, fl0, fl1 = carry
        pltpu.sync_copy(dst_hbm.at[pl.ds(b * B, B)], dstb)
        pltpu.sync_copy(src_hbm.at[pl.ds(b * B, B)], srcb)

        def scan_step(i, c):
            c0, c1 = c
            dv = dstb[pl.ds(i * LANE, LANE)]
            sv = srcb[pl.ds(i * LANE, LANE)]
            rel = dv - base
            m0 = (rel >= 0) & (rel < C)
            m1 = (rel >= C) & (rel < 2 * C)
            k0 = jnp.where(m0, lane, lane + LANE)
            cd0[pl.ds(c0, LANE)] = plsc.sort_key_val(k0, rel)[1]
            cs0[pl.ds(c0, LANE)] = plsc.sort_key_val(k0, sv)[1]
            k1 = jnp.where(m1, lane, lane + LANE)
            cd1[pl.ds(c1, LANE)] = plsc.sort_key_val(k1, rel - C)[1]
            cs1[pl.ds(c1, LANE)] = plsc.sort_key_val(k1, sv)[1]
            p0 = plsc.all_reduce_population_count(m0)
            p1 = plsc.all_reduce_population_count(m1)
            return (c0 + p0[0], c1 + p1[0])

        cnt0, cnt1 = lax.fori_loop(0, B // LANE, scan_step, (cnt0, cnt1))
        cnt0, fl0 = flush(cs0, cd0, cnt0, fl0, seg0)
        cnt1, fl1 = flush(cs1, cd1, cnt1, fl1, seg1)
        return (cnt0, cnt1, fl0, fl1)

    cnt0, cnt1, fl0, fl1 = lax.fori_loop(0, NB, block_step, (0, 0, 0, 0))

    for t in range(G // LANE):
        cs0[pl.ds(cnt0 + t * LANE, LANE)] = lane + (t * LANE)
        cs1[pl.ds(cnt1 + t * LANE, LANE)] = lane + (t * LANE)

    o0 = pl.multiple_of(seg0 + fl0, G)
    o1 = pl.multiple_of(seg1 + fl1, G)

    @pl.when(cnt0 > 0)
    def _t0():
        pltpu.sync_copy(cs0.at[pl.ds(0, G)], bsrc_hbm.at[pl.ds(o0, G)])
        pltpu.sync_copy(cd0.at[pl.ds(0, G)], bdst_hbm.at[pl.ds(o0, G)])

    @pl.when(cnt1 > 0)
    def _t1():
        pltpu.sync_copy(cs1.at[pl.ds(0, G)], bsrc_hbm.at[pl.ds(o1, G)])
        pltpu.sync_copy(cd1.at[pl.ds(0, G)], bdst_hbm.at[pl.ds(o1, G)])

    cv[pl.ds(0, LANE)] = jnp.broadcast_to(fl0 + cnt0, (LANE,)).astype(jnp.int32)
    cv[pl.ds(LANE, LANE)] = jnp.broadcast_to(fl1 + cnt1, (LANE,)).astype(jnp.int32)

    @pl.loop(2, 8)
    def _z(t):
        cv[pl.ds(t * LANE, LANE)] = jnp.zeros((LANE,), jnp.int32)

    pltpu.sync_copy(cv, counts_hbm.at[pl.ds(wid * 128, 128)])


@functools.cache
def _bucket_call():
    return functools.partial(
        pl.kernel,
        out_type=[
            jax.ShapeDtypeStruct((_BKT,), jnp.int32),
            jax.ShapeDtypeStruct((_BKT,), jnp.int32),
            jax.ShapeDtypeStruct((NWORK * 128,), jnp.int32),
        ],
        mesh=plsc.VectorSubcoreMesh(core_axis_name="c", subcore_axis_name="s",
                                    num_cores=NCORE, num_subcores=NSUB),
        compiler_params=pltpu.CompilerParams(needs_layout_passes=False),
        scratch_types=[
            pltpu.VMEM((B,), jnp.int32),
            pltpu.VMEM((B,), jnp.int32),
            pltpu.VMEM((B + G,), jnp.int32),
            pltpu.VMEM((B + G,), jnp.int32),
            pltpu.VMEM((B + G,), jnp.int32),
            pltpu.VMEM((B + G,), jnp.int32),
            pltpu.VMEM((128,), jnp.int32),
        ],
    )(_bucket_body)


def _sc_agg_body(x_hbm, bsrc_hbm, bdst_hbm, counts_hbm,
                 s_hbm, q_hbm, mx_hbm, mn_hbm, dg_hbm,
                 acc_s, acc_q, acc_x, acc_n, acc_d,
                 csA, cdA, csB, cdB, rowsA, rowsB, cntv,
                 semA, semB):
    wid = lax.axis_index("s") * NCORE + lax.axis_index("c")
    zeros = jnp.zeros((LANE,), jnp.float32)
    ones = jnp.ones((LANE,), jnp.float32)
    neg = jnp.full((LANE,), -_BIG, jnp.float32)
    pos = jnp.full((LANE,), _BIG, jnp.float32)

    pltpu.sync_copy(counts_hbm.at[pl.ds(wid * 128, 128)], cntv)

    for ch in range(CPT):
        chunk = wid * CPT + ch
        base = chunk * C
        seg = wid * (CPT * E) + ch * E
        cntc = cntv[pl.ds(ch * LANE, LANE)][0]

        @pl.loop(0, C)
        def _init(r):
            for f in range(FV):
                sl = pl.ds(f * LANE, LANE)
                acc_s[r, sl] = zeros
                acc_q[r, sl] = zeros
                acc_x[r, sl] = neg
                acc_n[r, sl] = pos
            acc_d[r, pl.ds(0, LANE)] = zeros

        nb = (cntc + G - 1) // G

        def stage_fire(p, cs, cd, rows, sem):
            o = pl.multiple_of(seg + p * G, G)
            pltpu.sync_copy(bsrc_hbm.at[pl.ds(o, G)], cs.at[pl.ds(0, G)])
            pltpu.sync_copy(bdst_hbm.at[pl.ds(o, G)], cd.at[pl.ds(0, G)])
            pltpu.async_copy(x_hbm.at[cs.at[pl.ds(0, G)]], rows, sem)

        def consume(p, cs, cd, rows, sem):
            pltpu.make_async_copy(x_hbm.at[cs.at[pl.ds(0, G)]], rows, sem).wait()
            cn = jnp.minimum(cntc - p * G, G)

            @pl.loop(0, cn)
            def _edge(j):
                dl = cd[pl.ds(j, LANE)][0]
                for f in range(FV):
                    sl = pl.ds(f * LANE, LANE)
                    mv = rows[j, sl]
                    plsc.addupdate(acc_s.at[dl, sl], mv)
                    plsc.addupdate(acc_q.at[dl, sl], mv * mv)
                    acc_x[dl, sl] = jnp.maximum(acc_x[dl, sl], mv)
                    acc_n[dl, sl] = jnp.minimum(acc_n[dl, sl], mv)
                plsc.addupdate(acc_d.at[dl, pl.ds(0, LANE)], ones)

        @pl.when(nb > 0)
        def _p0():
            stage_fire(0, csA, cdA, rowsA, semA)

        @pl.when(nb > 1)
        def _p1():
            stage_fire(1, csB, cdB, rowsB, semB)

        @pl.loop(0, nb)
        def _batch(p):
            even = lax.rem(p, 2) == 0

            @pl.when(even)
            def _e():
                consume(p, csA, cdA, rowsA, semA)

                @pl.when(p + 2 < nb)
                def _n():
                    stage_fire(p + 2, csA, cdA, rowsA, semA)

            @pl.when(jnp.logical_not(even))
            def _o():
                consume(p, csB, cdB, rowsB, semB)

                @pl.when(p + 2 < nb)
                def _n():
                    stage_fire(p + 2, csB, cdB, rowsB, semB)

        pltpu.sync_copy(acc_s, s_hbm.at[pl.ds(base, C)])
        pltpu.sync_copy(acc_q, q_hbm.at[pl.ds(base, C)])
        pltpu.sync_copy(acc_x, mx_hbm.at[pl.ds(base, C)])
        pltpu.sync_copy(acc_n, mn_hbm.at[pl.ds(base, C)])
        for off, sz in ((0, G), (G, G), (2 * G, C - 2 * G)):
            @pl.loop(0, sz)
            def _expand(r):
                v = acc_d[off + r, pl.ds(0, LANE)]
                for f in range(FV):
                    rowsA[r, pl.ds(f * LANE, LANE)] = v
            pltpu.sync_copy(rowsA.at[pl.ds(0, sz)],
                            dg_hbm.at[pl.ds(base + off, sz)])


_stat = jax.ShapeDtypeStruct((NP, D), jnp.float32)


@functools.cache
def _sc_agg_call():
    return functools.partial(
        pl.kernel,
        out_type=[_stat] * 5,
        mesh=plsc.VectorSubcoreMesh(core_axis_name="c", subcore_axis_name="s",
                                    num_cores=NCORE, num_subcores=NSUB),
        compiler_params=pltpu.CompilerParams(needs_layout_passes=False),
        scratch_types=[
            pltpu.VMEM((C, D), jnp.float32),
            pltpu.VMEM((C, D), jnp.float32),
            pltpu.VMEM((C, D), jnp.float32),
            pltpu.VMEM((C, D), jnp.float32),
            pltpu.VMEM((C, LANE), jnp.float32),
            pltpu.VMEM((G + LANE,), jnp.int32),
            pltpu.VMEM((G + LANE,), jnp.int32),
            pltpu.VMEM((G + LANE,), jnp.int32),
            pltpu.VMEM((G + LANE,), jnp.int32),
            pltpu.VMEM((G, D), jnp.float32),
            pltpu.VMEM((G, D), jnp.float32),
            pltpu.VMEM((128,), jnp.int32),
            pltpu.SemaphoreType.DMA,
            pltpu.SemaphoreType.DMA,
        ],
    )(_sc_agg_body)


def _scalers(sb, qb, xb, nb, db):
    degc = jnp.maximum(db, 1.0)
    mean = sb / degc
    var = jnp.maximum(qb / degc - mean * mean, 0.0)
    std = jnp.sqrt(var + 1e-5)
    has = db > 0.0
    mxm = jnp.where(has, xb, 0.0)
    mnm = jnp.where(has, nb, 0.0)
    logd = jnp.log(db + 1.0)
    amp = logd / AVG_D_LOG
    att = jnp.where(has, AVG_D_LOG / jnp.maximum(logd, 1e-6), 1.0)
    agg = jnp.concatenate([mean, mxm, mnm, std], axis=1)
    ampt = jnp.concatenate([amp] * 4, axis=1)
    attt = jnp.concatenate([att] * 4, axis=1)
    return agg, ampt, attt


def _mm_body(s_ref, q_ref, x_ref, n_ref, d_ref, w_ref, b_ref,
             out_ref, cs_ref, cq_ref):
    i = pl.program_id(0)
    agg, ampt, attt = _scalers(s_ref[...], q_ref[...], x_ref[...], n_ref[...],
                               d_ref[...])
    w = w_ref[...]
    out = jnp.dot(agg, w[0:4 * D], preferred_element_type=jnp.float32)
    out += jnp.dot(agg * ampt, w[4 * D:8 * D], preferred_element_type=jnp.float32)
    out += jnp.dot(agg * attt, w[8 * D:12 * D], preferred_element_type=jnp.float32)
    out += b_ref[...]
    out_ref[...] = out

    @pl.when(i == 0)
    def _():
        cs_ref[...] = jnp.zeros_like(cs_ref)
        cq_ref[...] = jnp.zeros_like(cq_ref)

    rid = lax.broadcasted_iota(jnp.int32, (R, D), 0) + i * R
    om = jnp.where(rid < N, out, 0.0)
    cs_ref[...] += jnp.sum(om, axis=0, keepdims=True)
    cq_ref[...] += jnp.sum(om * om, axis=0, keepdims=True)


def _mm_last_body(s_ref, q_ref, x_ref, n_ref, d_ref, w_ref, b_ref, h_ref,
                  out_ref):
    agg, ampt, attt = _scalers(s_ref[...], q_ref[...], x_ref[...], n_ref[...],
                               d_ref[...])
    w = w_ref[...]
    out = jnp.dot(agg, w[0:4 * D], preferred_element_type=jnp.float32)
    out += jnp.dot(agg * ampt, w[4 * D:8 * D], preferred_element_type=jnp.float32)
    out += jnp.dot(agg * attt, w[8 * D:12 * D], preferred_element_type=jnp.float32)
    out_ref[...] = out + b_ref[...] + h_ref[...]


def _bn_body(out_ref, h_ref, cs_ref, cq_ref, g_ref, b_ref, new_ref):
    mu = cs_ref[...] / N
    var = cq_ref[...] / N - mu * mu
    inv = lax.rsqrt(var + 1e-5)
    y = (out_ref[...] - mu) * inv * g_ref[...] + b_ref[...]
    new_ref[...] = h_ref[...] + jnp.maximum(y, 0.0)


_row_spec = pl.BlockSpec((R, D), lambda i: (i, 0))
_full_w = pl.BlockSpec((12 * D, D), lambda i: (0, 0))
_vec_spec = pl.BlockSpec((1, D), lambda i: (0, 0))

_mm_call = pl.pallas_call(
    _mm_body,
    grid=(NRB,),
    in_specs=[_row_spec] * 5 + [_full_w, _vec_spec],
    out_specs=[_row_spec, _vec_spec, _vec_spec],
    out_shape=[
        jax.ShapeDtypeStruct((NP, D), jnp.float32),
        jax.ShapeDtypeStruct((1, D), jnp.float32),
        jax.ShapeDtypeStruct((1, D), jnp.float32),
    ],
)

_mm_last_call = pl.pallas_call(
    _mm_last_body,
    grid=(NRB,),
    in_specs=[_row_spec] * 5 + [_full_w, _vec_spec, _row_spec],
    out_specs=_row_spec,
    out_shape=jax.ShapeDtypeStruct((NP, D), jnp.float32),
)

_bn_call = pl.pallas_call(
    _bn_body,
    grid=(NRB,),
    in_specs=[_row_spec, _row_spec, _vec_spec, _vec_spec, _vec_spec, _vec_spec],
    out_specs=_row_spec,
    out_shape=jax.ShapeDtypeStruct((NP, D), jnp.float32),
)


def kernel(h, e, W0, b0, W1, b1, W2, b2, W3, b3,
           gamma0, beta0, gamma1, beta1, gamma2, beta2, edge_index):
    del e
    src = edge_index[0]
    dst = edge_index[1]
    Ws = [W0, W1, W2, W3]
    bs = [b.reshape(1, D) for b in (b0, b1, b2, b3)]
    gammas = [g.reshape(1, D) for g in (gamma0, gamma1, gamma2)]
    betas = [b.reshape(1, D) for b in (beta0, beta1, beta2)]

    bsrc, bdst, counts = _bucket_call()(src, dst)
    x = jnp.pad(h, ((0, NP - N), (0, 0)))
    for i in range(4):
        s, q, mx, mn, dg = _sc_agg_call()(x, bsrc, bdst, counts)
        if i < 3:
            out, cs, cq = _mm_call(s, q, mx, mn, dg, Ws[i], bs[i])
            x = _bn_call(out, x, cs, cq, gammas[i], betas[i])
        else:
            x = _mm_last_call(s, q, mx, mn, dg, Ws[i], bs[i], x)
    return x[:N]


# speedup vs baseline: 62.4057x; 1.0670x over previous
import functools

import jax
import jax.numpy as jnp
from jax import lax
from jax.experimental import pallas as pl
from jax.experimental.pallas import tpu as pltpu
from jax.experimental.pallas import tpu_sc as plsc

N = 10000
E = 320000
D = 128
AVG_D_LOG = 3.4965

NCORE = 2
NSUB = 16
NWORK = NCORE * NSUB
LANE = 16
FV = D // LANE

C = 160
NCH = 64
NP = C * NCH
CPT = NCH // NWORK
B = 1280
NB = E // B
G = 64

R = 1024
NRB = NP // R

_BIG = 3.0e38

LB = 512
CAP = E + LB
_BKT = NCH * CAP


def _bucket_body(src_hbm, dst_hbm, busrc_hbm, budst_hbm, bsrc_hbm, bdst_hbm,
                 counts_hbm, dg_hbm,
                 srcb, dstb, cs0, cd0, cs1, cd1, cv, hist, lcs, lcd, idxb,
                 padi, padv, rowsd):
    wid = lax.axis_index("s") * NCORE + lax.axis_index("c")
    base = wid * (CPT * C)
    seg0 = wid * (CPT * CAP)
    seg1 = seg0 + CAP
    lane = lax.iota(jnp.int32, LANE)

    def flush(cs, cd, cnt, fl, seg):
        nf = cnt // G

        @pl.loop(0, nf)
        def _f(sb):
            o = pl.multiple_of(seg + fl + sb * G, G)
            pltpu.sync_copy(cs.at[pl.ds(sb * G, G)],
                            busrc_hbm.at[pl.ds(o, G)])
            pltpu.sync_copy(cd.at[pl.ds(sb * G, G)],
                            budst_hbm.at[pl.ds(o, G)])

        @pl.loop(0, G // LANE)
        def _m(t):
            o = nf * G + t * LANE
            cs[pl.ds(t * LANE, LANE)] = cs[pl.ds(o, LANE)]
            cd[pl.ds(t * LANE, LANE)] = cd[pl.ds(o, LANE)]

        return cnt - nf * G, fl + nf * G

    def block_step(b, carry):
        cnt0, cnt1, fl0, fl1 = carry
        pltpu.sync_copy(dst_hbm.at[pl.ds(b * B, B)], dstb)
        pltpu.sync_copy(src_hbm.at[pl.ds(b * B, B)], srcb)

        def scan_step(i, c):
            c0, c1 = c
            dv = dstb[pl.ds(i * LANE, LANE)]
            sv = srcb[pl.ds(i * LANE, LANE)]
            rel = dv - base
            m0 = (rel >= 0) & (rel < C)
            m1 = (rel >= C) & (rel < 2 * C)
            k0 = jnp.where(m0, lane, lane + LANE)
            cd0[pl.ds(c0, LANE)] = plsc.sort_key_val(k0, rel)[1]
            cs0[pl.ds(c0, LANE)] = plsc.sort_key_val(k0, sv)[1]
            k1 = jnp.where(m1, lane, lane + LANE)
            cd1[pl.ds(c1, LANE)] = plsc.sort_key_val(k1, rel - C)[1]
            cs1[pl.ds(c1, LANE)] = plsc.sort_key_val(k1, sv)[1]
            p0 = plsc.all_reduce_population_count(m0)
            p1 = plsc.all_reduce_population_count(m1)
            return (c0 + p0[0], c1 + p1[0])

        cnt0, cnt1 = lax.fori_loop(0, B // LANE, scan_step, (cnt0, cnt1))
        cnt0, fl0 = flush(cs0, cd0, cnt0, fl0, seg0)
        cnt1, fl1 = flush(cs1, cd1, cnt1, fl1, seg1)
        return (cnt0, cnt1, fl0, fl1)

    cnt0, cnt1, fl0, fl1 = lax.fori_loop(0, NB, block_step, (0, 0, 0, 0))

    for t in range(G // LANE):
        cs0[pl.ds(cnt0 + t * LANE, LANE)] = lane + (t * LANE)
        cs1[pl.ds(cnt1 + t * LANE, LANE)] = lane + (t * LANE)

    o0 = pl.multiple_of(seg0 + fl0, G)
    o1 = pl.multiple_of(seg1 + fl1, G)

    @pl.when(cnt0 > 0)
    def _t0():
        pltpu.sync_copy(cs0.at[pl.ds(0, G)], busrc_hbm.at[pl.ds(o0, G)])
        pltpu.sync_copy(cd0.at[pl.ds(0, G)], budst_hbm.at[pl.ds(o0, G)])

    @pl.when(cnt1 > 0)
    def _t1():
        pltpu.sync_copy(cs1.at[pl.ds(0, G)], busrc_hbm.at[pl.ds(o1, G)])
        pltpu.sync_copy(cd1.at[pl.ds(0, G)], budst_hbm.at[pl.ds(o1, G)])

    cv[pl.ds(0, LANE)] = jnp.broadcast_to(fl0 + cnt0, (LANE,)).astype(jnp.int32)
    cv[pl.ds(LANE, LANE)] = jnp.broadcast_to(fl1 + cnt1, (LANE,)).astype(jnp.int32)

    @pl.loop(2, 8)
    def _z(t):
        cv[pl.ds(t * LANE, LANE)] = jnp.zeros((LANE,), jnp.int32)

    pltpu.sync_copy(cv, counts_hbm.at[pl.ds(wid * 128, 128)])

    izeros = jnp.zeros((LANE,), jnp.int32)
    iones = jnp.ones((LANE,), jnp.int32)
    totals = (fl0 + cnt0, fl1 + cnt1)
    for ch in range(CPT):
        chunk = wid * CPT + ch
        seg = seg0 + ch * CAP
        cnt = totals[ch]

        @pl.loop(0, C)
        def _hinit(r):
            hist[r, pl.ds(0, LANE)] = izeros

        nlb = (cnt + LB - 1) // LB

        @pl.loop(0, nlb)
        def _h(bb):
            o = pl.multiple_of(seg + bb * LB, G)
            pltpu.sync_copy(budst_hbm.at[pl.ds(o, LB)], lcd)
            cnb = jnp.minimum(cnt - bb * LB, LB)

            @pl.loop(0, cnb)
            def _e(j):
                dl = lcd[pl.ds(j, LANE)][0]
                hist[dl, pl.ds(0, LANE)] = hist[dl, pl.ds(0, LANE)] + iones

        nodebase = chunk * C
        for off, sz in ((0, G), (G, G), (2 * G, C - 2 * G)):
            @pl.loop(0, sz)
            def _dx(r):
                v = hist[off + r, pl.ds(0, LANE)].astype(jnp.float32)
                for f in range(FV):
                    rowsd[r, pl.ds(f * LANE, LANE)] = v
            pltpu.sync_copy(rowsd.at[pl.ds(0, sz)],
                            dg_hbm.at[pl.ds(nodebase + off, sz)])

        def _pref(r, run):
            hv = hist[r, pl.ds(0, LANE)][0]
            hist[r, pl.ds(0, LANE)] = jnp.broadcast_to(run, (LANE,)).astype(jnp.int32)
            return run + hv

        _ = lax.fori_loop(0, C, _pref, seg)

        lane0 = lane == 0
        izero16 = jnp.zeros((LANE,), jnp.int32)

        @pl.loop(0, nlb)
        def _p(bb):
            o = pl.multiple_of(seg + bb * LB, G)
            pltpu.sync_copy(busrc_hbm.at[pl.ds(o, LB)], lcs)
            pltpu.sync_copy(budst_hbm.at[pl.ds(o, LB)], lcd)
            cnb = jnp.minimum(cnt - bb * LB, LB)

            @pl.when(cnb < LB)
            def _fill():
                @pl.loop(0, LB // LANE)
                def _ft(t):
                    idxb[0, pl.ds(t * LANE, LANE)] = (
                        jnp.broadcast_to(seg + cnt, (LANE,)).astype(jnp.int32)
                        + lane + t * LANE)

            @pl.loop(0, cnb)
            def _e(j):
                dl = lcd[pl.ds(j, LANE)][0]
                hrow = hist[dl, pl.ds(0, LANE)]
                plsc.store_scatter(
                    idxb,
                    [izero16, jnp.broadcast_to(j, (LANE,)).astype(jnp.int32)],
                    hrow, mask=lane0)
                hist[dl, pl.ds(0, LANE)] = hrow + 1

            pltpu.sync_copy(lcs, bsrc_hbm.at[idxb.at[0]])
            pltpu.sync_copy(lcd, bdst_hbm.at[idxb.at[0]])

        for t in range(G // LANE):
            padi[0, pl.ds(t * LANE, LANE)] = (
                jnp.broadcast_to(seg + cnt, (LANE,)).astype(jnp.int32)
                + lane + t * LANE)
            padv[pl.ds(t * LANE, LANE)] = lane + t * LANE
        pltpu.sync_copy(padv, bsrc_hbm.at[padi.at[0]])


@functools.cache
def _bucket_call():
    return functools.partial(
        pl.kernel,
        out_type=[
            jax.ShapeDtypeStruct((_BKT,), jnp.int32),
            jax.ShapeDtypeStruct((_BKT,), jnp.int32),
            jax.ShapeDtypeStruct((_BKT,), jnp.int32),
            jax.ShapeDtypeStruct((_BKT,), jnp.int32),
            jax.ShapeDtypeStruct((NWORK * 128,), jnp.int32),
            jax.ShapeDtypeStruct((NP, D), jnp.float32),
        ],
        mesh=plsc.VectorSubcoreMesh(core_axis_name="c", subcore_axis_name="s",
                                    num_cores=NCORE, num_subcores=NSUB),
        compiler_params=pltpu.CompilerParams(needs_layout_passes=False),
        scratch_types=[
            pltpu.VMEM((B,), jnp.int32),
            pltpu.VMEM((B,), jnp.int32),
            pltpu.VMEM((B + G,), jnp.int32),
            pltpu.VMEM((B + G,), jnp.int32),
            pltpu.VMEM((B + G,), jnp.int32),
            pltpu.VMEM((B + G,), jnp.int32),
            pltpu.VMEM((128,), jnp.int32),
            pltpu.VMEM((C, LANE), jnp.int32),
            pltpu.VMEM((LB,), jnp.int32),
            pltpu.VMEM((LB,), jnp.int32),
            pltpu.VMEM((1, LB), jnp.int32),
            pltpu.VMEM((1, G), jnp.int32),
            pltpu.VMEM((G,), jnp.int32),
            pltpu.VMEM((G, D), jnp.float32),
        ],
    )(_bucket_body)


def _sc_agg_body(x_hbm, bsrc_hbm, bdst_hbm, counts_hbm,
                 s_hbm, q_hbm, mx_hbm, mn_hbm,
                 acc_s, acc_q, acc_x, acc_n,
                 csA, cdA, csB, cdB, rowsA, rowsB, cntv,
                 semA, semB):
    wid = lax.axis_index("s") * NCORE + lax.axis_index("c")
    zeros = jnp.zeros((LANE,), jnp.float32)
    neg = jnp.full((LANE,), -_BIG, jnp.float32)
    pos = jnp.full((LANE,), _BIG, jnp.float32)

    pltpu.sync_copy(counts_hbm.at[pl.ds(wid * 128, 128)], cntv)

    for ch in range(CPT):
        chunk = wid * CPT + ch
        base = chunk * C
        seg = wid * (CPT * CAP) + ch * CAP
        cntc = cntv[pl.ds(ch * LANE, LANE)][0]

        @pl.loop(0, C)
        def _init(r):
            for f in range(FV):
                sl = pl.ds(f * LANE, LANE)
                acc_s[r, sl] = zeros
                acc_q[r, sl] = zeros

        nb = (cntc + G - 1) // G

        def stage_fire(p, cs, cd, rows, sem):
            o = pl.multiple_of(seg + p * G, G)
            pltpu.sync_copy(bsrc_hbm.at[pl.ds(o, G)], cs.at[pl.ds(0, G)])
            pltpu.sync_copy(bdst_hbm.at[pl.ds(o, G)], cd.at[pl.ds(0, G)])
            pltpu.async_copy(x_hbm.at[cs.at[pl.ds(0, G)]], rows, sem)

        def flush(run, s, q, x, n):
            for f in range(FV):
                sl = pl.ds(f * LANE, LANE)
                acc_s[run, sl] = s[f]
                acc_q[run, sl] = q[f]
                acc_x[run, sl] = x[f]
                acc_n[run, sl] = n[f]

        def consume(p, cs, cd, rows, sem, carry):
            @pl.when(p < nb)
            def _w():
                pltpu.make_async_copy(x_hbm.at[cs.at[pl.ds(0, G)]], rows,
                                      sem).wait()

            cn = jnp.minimum(jnp.maximum(cntc - p * G, 0), G)

            def edge(j, c):
                run, s, q, x, n = c
                dl = cd[pl.ds(j, LANE)][0]
                change = dl != run

                @pl.when(change & (run >= 0))
                def _fl():
                    flush(run, s, q, x, n)

                ns, nq, nx, nn = [], [], [], []
                for f in range(FV):
                    mv = rows[j, pl.ds(f * LANE, LANE)]
                    ns.append(jnp.where(change, zeros, s[f]) + mv)
                    nq.append(jnp.where(change, zeros, q[f]) + mv * mv)
                    nx.append(jnp.maximum(jnp.where(change, neg, x[f]), mv))
                    nn.append(jnp.minimum(jnp.where(change, pos, n[f]), mv))
                return (dl, tuple(ns), tuple(nq), tuple(nx), tuple(nn))

            carry = lax.fori_loop(0, cn, edge, carry)

            @pl.when(p + 2 < nb)
            def _f():
                stage_fire(p + 2, cs, cd, rows, sem)

            return carry

        @pl.when(nb > 0)
        def _p0():
            stage_fire(0, csA, cdA, rowsA, semA)

        @pl.when(nb > 1)
        def _p1():
            stage_fire(1, csB, cdB, rowsB, semB)

        def pair(t, carry):
            carry = consume(2 * t, csA, cdA, rowsA, semA, carry)
            carry = consume(2 * t + 1, csB, cdB, rowsB, semB, carry)
            return carry

        init = (jnp.asarray(-1, jnp.int32),
                (zeros,) * FV, (zeros,) * FV, (neg,) * FV, (pos,) * FV)
        run, s, q, x, n = lax.fori_loop(0, (nb + 1) // 2, pair, init)

        @pl.when(run >= 0)
        def _last():
            flush(run, s, q, x, n)

        pltpu.sync_copy(acc_s, s_hbm.at[pl.ds(base, C)])
        pltpu.sync_copy(acc_q, q_hbm.at[pl.ds(base, C)])
        pltpu.sync_copy(acc_x, mx_hbm.at[pl.ds(base, C)])
        pltpu.sync_copy(acc_n, mn_hbm.at[pl.ds(base, C)])


_stat = jax.ShapeDtypeStruct((NP, D), jnp.float32)


@functools.cache
def _sc_agg_call():
    return functools.partial(
        pl.kernel,
        out_type=[_stat] * 4,
        mesh=plsc.VectorSubcoreMesh(core_axis_name="c", subcore_axis_name="s",
                                    num_cores=NCORE, num_subcores=NSUB),
        compiler_params=pltpu.CompilerParams(needs_layout_passes=False),
        scratch_types=[
            pltpu.VMEM((C, D), jnp.float32),
            pltpu.VMEM((C, D), jnp.float32),
            pltpu.VMEM((C, D), jnp.float32),
            pltpu.VMEM((C, D), jnp.float32),
            pltpu.VMEM((G + LANE,), jnp.int32),
            pltpu.VMEM((G + LANE,), jnp.int32),
            pltpu.VMEM((G + LANE,), jnp.int32),
            pltpu.VMEM((G + LANE,), jnp.int32),
            pltpu.VMEM((G, D), jnp.float32),
            pltpu.VMEM((G, D), jnp.float32),
            pltpu.VMEM((128,), jnp.int32),
            pltpu.SemaphoreType.DMA,
            pltpu.SemaphoreType.DMA,
        ],
    )(_sc_agg_body)


def _scalers(sb, qb, xb, nb, db):
    degc = jnp.maximum(db, 1.0)
    mean = sb / degc
    var = jnp.maximum(qb / degc - mean * mean, 0.0)
    std = jnp.sqrt(var + 1e-5)
    has = db > 0.0
    mxm = jnp.where(has, xb, 0.0)
    mnm = jnp.where(has, nb, 0.0)
    logd = jnp.log(db + 1.0)
    amp = logd / AVG_D_LOG
    att = jnp.where(has, AVG_D_LOG / jnp.maximum(logd, 1e-6), 1.0)
    agg = jnp.concatenate([mean, mxm, mnm, std], axis=1)
    ampt = jnp.concatenate([amp] * 4, axis=1)
    attt = jnp.concatenate([att] * 4, axis=1)
    return agg, ampt, attt


def _mm_body(s_ref, q_ref, x_ref, n_ref, d_ref, w_ref, b_ref,
             out_ref, cs_ref, cq_ref):
    i = pl.program_id(0)
    agg, ampt, attt = _scalers(s_ref[...], q_ref[...], x_ref[...], n_ref[...],
                               d_ref[...])
    w = w_ref[...]
    out = jnp.dot(agg, w[0:4 * D], preferred_element_type=jnp.float32)
    out += jnp.dot(agg * ampt, w[4 * D:8 * D], preferred_element_type=jnp.float32)
    out += jnp.dot(agg * attt, w[8 * D:12 * D], preferred_element_type=jnp.float32)
    out += b_ref[...]
    out_ref[...] = out

    @pl.when(i == 0)
    def _():
        cs_ref[...] = jnp.zeros_like(cs_ref)
        cq_ref[...] = jnp.zeros_like(cq_ref)

    rid = lax.broadcasted_iota(jnp.int32, (R, D), 0) + i * R
    om = jnp.where(rid < N, out, 0.0)
    cs_ref[...] += jnp.sum(om, axis=0, keepdims=True)
    cq_ref[...] += jnp.sum(om * om, axis=0, keepdims=True)


def _mm_last_body(s_ref, q_ref, x_ref, n_ref, d_ref, w_ref, b_ref, h_ref,
                  out_ref):
    agg, ampt, attt = _scalers(s_ref[...], q_ref[...], x_ref[...], n_ref[...],
                               d_ref[...])
    w = w_ref[...]
    out = jnp.dot(agg, w[0:4 * D], preferred_element_type=jnp.float32)
    out += jnp.dot(agg * ampt, w[4 * D:8 * D], preferred_element_type=jnp.float32)
    out += jnp.dot(agg * attt, w[8 * D:12 * D], preferred_element_type=jnp.float32)
    out_ref[...] = out + b_ref[...] + h_ref[...]


def _bn_body(out_ref, h_ref, cs_ref, cq_ref, g_ref, b_ref, new_ref):
    mu = cs_ref[...] / N
    var = cq_ref[...] / N - mu * mu
    inv = lax.rsqrt(var + 1e-5)
    y = (out_ref[...] - mu) * inv * g_ref[...] + b_ref[...]
    new_ref[...] = h_ref[...] + jnp.maximum(y, 0.0)


_row_spec = pl.BlockSpec((R, D), lambda i: (i, 0))
_full_w = pl.BlockSpec((12 * D, D), lambda i: (0, 0))
_vec_spec = pl.BlockSpec((1, D), lambda i: (0, 0))

_mm_call = pl.pallas_call(
    _mm_body,
    grid=(NRB,),
    in_specs=[_row_spec] * 5 + [_full_w, _vec_spec],
    out_specs=[_row_spec, _vec_spec, _vec_spec],
    out_shape=[
        jax.ShapeDtypeStruct((NP, D), jnp.float32),
        jax.ShapeDtypeStruct((1, D), jnp.float32),
        jax.ShapeDtypeStruct((1, D), jnp.float32),
    ],
)

_mm_last_call = pl.pallas_call(
    _mm_last_body,
    grid=(NRB,),
    in_specs=[_row_spec] * 5 + [_full_w, _vec_spec, _row_spec],
    out_specs=_row_spec,
    out_shape=jax.ShapeDtypeStruct((NP, D), jnp.float32),
)

_bn_call = pl.pallas_call(
    _bn_body,
    grid=(NRB,),
    in_specs=[_row_spec, _row_spec, _vec_spec, _vec_spec, _vec_spec, _vec_spec],
    out_specs=_row_spec,
    out_shape=jax.ShapeDtypeStruct((NP, D), jnp.float32),
)


def kernel(h, e, W0, b0, W1, b1, W2, b2, W3, b3,
           gamma0, beta0, gamma1, beta1, gamma2, beta2, edge_index):
    del e
    src = edge_index[0]
    dst = edge_index[1]
    Ws = [W0, W1, W2, W3]
    bs = [b.reshape(1, D) for b in (b0, b1, b2, b3)]
    gammas = [g.reshape(1, D) for g in (gamma0, gamma1, gamma2)]
    betas = [b.reshape(1, D) for b in (beta0, beta1, beta2)]

    _, _, bsrc, bdst, counts, dg = _bucket_call()(src, dst)
    x = jnp.pad(h, ((0, NP - N), (0, 0)))
    for i in range(4):
        s, q, mx, mn = _sc_agg_call()(x, bsrc, bdst, counts)
        if i < 3:
            out, cs, cq = _mm_call(s, q, mx, mn, dg, Ws[i], bs[i])
            x = _bn_call(out, x, cs, cq, gammas[i], betas[i])
        else:
            x = _mm_last_call(s, q, mx, mn, dg, Ws[i], bs[i], x)
    return x[:N]


# speedup vs baseline: 115.9715x; 1.8583x over previous
import functools

import jax
import jax.numpy as jnp
from jax import lax
from jax.experimental import pallas as pl
from jax.experimental.pallas import tpu as pltpu
from jax.experimental.pallas import tpu_sc as plsc

N = 10000
E = 320000
D = 128
AVG_D_LOG = 3.4965

NCORE = 2
NSUB = 16
NWORK = NCORE * NSUB
LANE = 16
FV = D // LANE

C = 160
NCH = 64
NP = C * NCH
CPT = NCH // NWORK
B = 1280
NB = E // B
G = 64

R = 1024
NRB = NP // R

_BIG = 3.0e38

LB = 512
CAP = E + LB
_BKT = NCH * CAP


def _bucket_body(src_hbm, dst_hbm, busrc_hbm, budst_hbm, bsrc_hbm,
                 counts_hbm, dg_hbm, deg_hbm,
                 srcb, dstb, cs0, cd0, cs1, cd1, cv, hist, lcs, lcd, idxb,
                 padi, padv, rowsd):
    wid = lax.axis_index("s") * NCORE + lax.axis_index("c")
    base = wid * (CPT * C)
    seg0 = wid * (CPT * CAP)
    seg1 = seg0 + CAP
    lane = lax.iota(jnp.int32, LANE)

    def flush(cs, cd, cnt, fl, seg):
        nf = cnt // G

        @pl.loop(0, nf)
        def _f(sb):
            o = pl.multiple_of(seg + fl + sb * G, G)
            pltpu.sync_copy(cs.at[pl.ds(sb * G, G)],
                            busrc_hbm.at[pl.ds(o, G)])
            pltpu.sync_copy(cd.at[pl.ds(sb * G, G)],
                            budst_hbm.at[pl.ds(o, G)])

        @pl.loop(0, G // LANE)
        def _m(t):
            o = nf * G + t * LANE
            cs[pl.ds(t * LANE, LANE)] = cs[pl.ds(o, LANE)]
            cd[pl.ds(t * LANE, LANE)] = cd[pl.ds(o, LANE)]

        return cnt - nf * G, fl + nf * G

    def block_step(b, carry):
        cnt0, cnt1, fl0, fl1 = carry
        pltpu.sync_copy(dst_hbm.at[pl.ds(b * B, B)], dstb)
        pltpu.sync_copy(src_hbm.at[pl.ds(b * B, B)], srcb)

        def scan_step(i, c):
            c0, c1 = c
            dv = dstb[pl.ds(i * LANE, LANE)]
            sv = srcb[pl.ds(i * LANE, LANE)]
            rel = dv - base
            m0 = (rel >= 0) & (rel < C)
            m1 = (rel >= C) & (rel < 2 * C)
            k0 = jnp.where(m0, lane, lane + LANE)
            cd0[pl.ds(c0, LANE)] = plsc.sort_key_val(k0, rel)[1]
            cs0[pl.ds(c0, LANE)] = plsc.sort_key_val(k0, sv)[1]
            k1 = jnp.where(m1, lane, lane + LANE)
            cd1[pl.ds(c1, LANE)] = plsc.sort_key_val(k1, rel - C)[1]
            cs1[pl.ds(c1, LANE)] = plsc.sort_key_val(k1, sv)[1]
            p0 = plsc.all_reduce_population_count(m0)
            p1 = plsc.all_reduce_population_count(m1)
            return (c0 + p0[0], c1 + p1[0])

        cnt0, cnt1 = lax.fori_loop(0, B // LANE, scan_step, (cnt0, cnt1))
        cnt0, fl0 = flush(cs0, cd0, cnt0, fl0, seg0)
        cnt1, fl1 = flush(cs1, cd1, cnt1, fl1, seg1)
        return (cnt0, cnt1, fl0, fl1)

    cnt0, cnt1, fl0, fl1 = lax.fori_loop(0, NB, block_step, (0, 0, 0, 0))

    for t in range(G // LANE):
        cs0[pl.ds(cnt0 + t * LANE, LANE)] = lane + (t * LANE)
        cs1[pl.ds(cnt1 + t * LANE, LANE)] = lane + (t * LANE)

    o0 = pl.multiple_of(seg0 + fl0, G)
    o1 = pl.multiple_of(seg1 + fl1, G)

    @pl.when(cnt0 > 0)
    def _t0():
        pltpu.sync_copy(cs0.at[pl.ds(0, G)], busrc_hbm.at[pl.ds(o0, G)])
        pltpu.sync_copy(cd0.at[pl.ds(0, G)], budst_hbm.at[pl.ds(o0, G)])

    @pl.when(cnt1 > 0)
    def _t1():
        pltpu.sync_copy(cs1.at[pl.ds(0, G)], busrc_hbm.at[pl.ds(o1, G)])
        pltpu.sync_copy(cd1.at[pl.ds(0, G)], budst_hbm.at[pl.ds(o1, G)])

    cv[pl.ds(0, LANE)] = jnp.broadcast_to(fl0 + cnt0, (LANE,)).astype(jnp.int32)
    cv[pl.ds(LANE, LANE)] = jnp.broadcast_to(fl1 + cnt1, (LANE,)).astype(jnp.int32)

    @pl.loop(2, 8)
    def _z(t):
        cv[pl.ds(t * LANE, LANE)] = jnp.zeros((LANE,), jnp.int32)

    pltpu.sync_copy(cv, counts_hbm.at[pl.ds(wid * 128, 128)])

    izeros = jnp.zeros((LANE,), jnp.int32)
    iones = jnp.ones((LANE,), jnp.int32)
    totals = (fl0 + cnt0, fl1 + cnt1)
    for ch in range(CPT):
        chunk = wid * CPT + ch
        seg = seg0 + ch * CAP
        cnt = totals[ch]

        @pl.loop(0, C)
        def _hinit(r):
            hist[r, pl.ds(0, LANE)] = izeros

        nlb = (cnt + LB - 1) // LB

        @pl.loop(0, nlb)
        def _h(bb):
            o = pl.multiple_of(seg + bb * LB, G)
            pltpu.sync_copy(budst_hbm.at[pl.ds(o, LB)], lcd)
            cnb = jnp.minimum(cnt - bb * LB, LB)

            @pl.loop(0, cnb)
            def _e(j):
                dl = lcd[pl.ds(j, LANE)][0]
                hist[dl, pl.ds(0, LANE)] = hist[dl, pl.ds(0, LANE)] + iones

        nodebase = chunk * C
        for off, sz in ((0, G), (G, G), (2 * G, C - 2 * G)):
            @pl.loop(0, sz)
            def _dx(r):
                v = hist[off + r, pl.ds(0, LANE)].astype(jnp.float32)
                for f in range(FV):
                    rowsd[r, pl.ds(f * LANE, LANE)] = v
            pltpu.sync_copy(rowsd.at[pl.ds(0, sz)],
                            dg_hbm.at[pl.ds(nodebase + off, sz)])
        pltpu.sync_copy(hist, deg_hbm.at[pl.ds(nodebase, C)])

        def _pref(r, run):
            hv = hist[r, pl.ds(0, LANE)][0]
            hist[r, pl.ds(0, LANE)] = jnp.broadcast_to(run, (LANE,)).astype(jnp.int32)
            return run + hv

        _ = lax.fori_loop(0, C, _pref, seg)

        lane0 = lane == 0
        izero16 = jnp.zeros((LANE,), jnp.int32)

        @pl.loop(0, nlb)
        def _p(bb):
            o = pl.multiple_of(seg + bb * LB, G)
            pltpu.sync_copy(busrc_hbm.at[pl.ds(o, LB)], lcs)
            pltpu.sync_copy(budst_hbm.at[pl.ds(o, LB)], lcd)
            cnb = jnp.minimum(cnt - bb * LB, LB)

            @pl.when(cnb < LB)
            def _fill():
                @pl.loop(0, LB // LANE)
                def _ft(t):
                    idxb[0, pl.ds(t * LANE, LANE)] = (
                        jnp.broadcast_to(seg + cnt, (LANE,)).astype(jnp.int32)
                        + lane + t * LANE)

            @pl.loop(0, cnb)
            def _e(j):
                dl = lcd[pl.ds(j, LANE)][0]
                hrow = hist[dl, pl.ds(0, LANE)]
                plsc.store_scatter(
                    idxb,
                    [izero16, jnp.broadcast_to(j, (LANE,)).astype(jnp.int32)],
                    hrow, mask=lane0)
                hist[dl, pl.ds(0, LANE)] = hrow + 1

            pltpu.sync_copy(lcs, bsrc_hbm.at[idxb.at[0]])

        for t in range(G // LANE):
            padi[0, pl.ds(t * LANE, LANE)] = (
                jnp.broadcast_to(seg + cnt, (LANE,)).astype(jnp.int32)
                + lane + t * LANE)
            padv[pl.ds(t * LANE, LANE)] = lane + t * LANE
        pltpu.sync_copy(padv, bsrc_hbm.at[padi.at[0]])


@functools.cache
def _bucket_call():
    return functools.partial(
        pl.kernel,
        out_type=[
            jax.ShapeDtypeStruct((_BKT,), jnp.int32),
            jax.ShapeDtypeStruct((_BKT,), jnp.int32),
            jax.ShapeDtypeStruct((_BKT,), jnp.int32),
            jax.ShapeDtypeStruct((NWORK * 128,), jnp.int32),
            jax.ShapeDtypeStruct((NP, D), jnp.float32),
            jax.ShapeDtypeStruct((NCH * C, LANE), jnp.int32),
        ],
        mesh=plsc.VectorSubcoreMesh(core_axis_name="c", subcore_axis_name="s",
                                    num_cores=NCORE, num_subcores=NSUB),
        compiler_params=pltpu.CompilerParams(needs_layout_passes=False),
        scratch_types=[
            pltpu.VMEM((B,), jnp.int32),
            pltpu.VMEM((B,), jnp.int32),
            pltpu.VMEM((B + G,), jnp.int32),
            pltpu.VMEM((B + G,), jnp.int32),
            pltpu.VMEM((B + G,), jnp.int32),
            pltpu.VMEM((B + G,), jnp.int32),
            pltpu.VMEM((128,), jnp.int32),
            pltpu.VMEM((C, LANE), jnp.int32),
            pltpu.VMEM((LB,), jnp.int32),
            pltpu.VMEM((LB,), jnp.int32),
            pltpu.VMEM((1, LB), jnp.int32),
            pltpu.VMEM((1, G), jnp.int32),
            pltpu.VMEM((G,), jnp.int32),
            pltpu.VMEM((G, D), jnp.float32),
        ],
    )(_bucket_body)


def _sc_agg_body(x_hbm, bsrc_hbm, deg_hbm, counts_hbm,
                 s_hbm, q_hbm, mx_hbm, mn_hbm,
                 acc_s, acc_q, acc_x, acc_n,
                 csA, csB, rowsA, rowsB, cntv, degv,
                 semA, semB):
    wid = lax.axis_index("s") * NCORE + lax.axis_index("c")
    zeros = jnp.zeros((LANE,), jnp.float32)
    neg = jnp.full((LANE,), -_BIG, jnp.float32)
    pos = jnp.full((LANE,), _BIG, jnp.float32)

    pltpu.sync_copy(counts_hbm.at[pl.ds(wid * 128, 128)], cntv)

    for ch in range(CPT):
        chunk = wid * CPT + ch
        base = chunk * C
        seg = wid * (CPT * CAP) + ch * CAP
        cntc = cntv[pl.ds(ch * LANE, LANE)][0]

        pltpu.sync_copy(deg_hbm.at[pl.ds(base, C)], degv.at[pl.ds(0, C)])

        @pl.loop(0, C)
        def _init(r):
            for f in range(FV):
                sl = pl.ds(f * LANE, LANE)
                acc_s[r, sl] = zeros
                acc_q[r, sl] = zeros

        nb = (cntc + G - 1) // G

        def stage_fire(p, cs, rows, sem):
            o = pl.multiple_of(seg + p * G, G)
            pltpu.sync_copy(bsrc_hbm.at[pl.ds(o, G)], cs.at[pl.ds(0, G)])
            pltpu.async_copy(x_hbm.at[cs.at[pl.ds(0, G)]], rows, sem)

        def flush(r, s, q, x, n):
            for f in range(FV):
                sl = pl.ds(f * LANE, LANE)
                acc_s[r, sl] = s[f]
                acc_q[r, sl] = q[f]
                acc_x[r, sl] = x[f]
                acc_n[r, sl] = n[f]

        def consume(p, cs, rows, sem, carry):
            @pl.when(p < nb)
            def _w():
                pltpu.make_async_copy(x_hbm.at[cs.at[pl.ds(0, G)]], rows,
                                      sem).wait()

            cn = jnp.minimum(jnp.maximum(cntc - p * G, 0), G)

            def run_cond(st):
                return st[0] < cn

            def run_body(st):
                j, r, rem, s, q, x, n = st
                take = jnp.minimum(rem, cn - j)

                def ek(k, c):
                    s, q, x, n = c
                    ns, nq, nx, nn = [], [], [], []
                    for f in range(FV):
                        mv = rows[k, pl.ds(f * LANE, LANE)]
                        ns.append(s[f] + mv)
                        nq.append(q[f] + mv * mv)
                        nx.append(jnp.maximum(x[f], mv))
                        nn.append(jnp.minimum(n[f], mv))
                    return (tuple(ns), tuple(nq), tuple(nx), tuple(nn))

                s, q, x, n = lax.fori_loop(j, j + take, ek, (s, q, x, n))
                j = j + take
                rem = rem - take
                done = rem == 0

                @pl.when(done)
                def _fl():
                    flush(r, s, q, x, n)

                nr = jnp.where(done, r + 1, r)
                nrem = jnp.where(done, degv[nr, pl.ds(0, LANE)][0], rem)
                sel = lambda a, b: tuple(jnp.where(done, a, v) for v in b)
                return (j, nr, nrem, sel(zeros, s), sel(zeros, q),
                        sel(neg, x), sel(pos, n))

            carry = lax.while_loop(run_cond, run_body, (jnp.asarray(0, jnp.int32),) + carry)[1:]

            @pl.when(p + 2 < nb)
            def _f():
                stage_fire(p + 2, cs, rows, sem)

            return carry

        @pl.when(nb > 0)
        def _p0():
            stage_fire(0, csA, rowsA, semA)

        @pl.when(nb > 1)
        def _p1():
            stage_fire(1, csB, rowsB, semB)

        def pair(t, carry):
            carry = consume(2 * t, csA, rowsA, semA, carry)
            carry = consume(2 * t + 1, csB, rowsB, semB, carry)
            return carry

        init = (jnp.asarray(0, jnp.int32), degv[0, pl.ds(0, LANE)][0],
                (zeros,) * FV, (zeros,) * FV, (neg,) * FV, (pos,) * FV)
        lax.fori_loop(0, (nb + 1) // 2, pair, init)

        pltpu.sync_copy(acc_s, s_hbm.at[pl.ds(base, C)])
        pltpu.sync_copy(acc_q, q_hbm.at[pl.ds(base, C)])
        pltpu.sync_copy(acc_x, mx_hbm.at[pl.ds(base, C)])
        pltpu.sync_copy(acc_n, mn_hbm.at[pl.ds(base, C)])


_stat = jax.ShapeDtypeStruct((NP, D), jnp.float32)


@functools.cache
def _sc_agg_call():
    return functools.partial(
        pl.kernel,
        out_type=[_stat] * 4,
        mesh=plsc.VectorSubcoreMesh(core_axis_name="c", subcore_axis_name="s",
                                    num_cores=NCORE, num_subcores=NSUB),
        compiler_params=pltpu.CompilerParams(needs_layout_passes=False),
        scratch_types=[
            pltpu.VMEM((C, D), jnp.float32),
            pltpu.VMEM((C, D), jnp.float32),
            pltpu.VMEM((C, D), jnp.float32),
            pltpu.VMEM((C, D), jnp.float32),
            pltpu.VMEM((G + LANE,), jnp.int32),
            pltpu.VMEM((G + LANE,), jnp.int32),
            pltpu.VMEM((G, D), jnp.float32),
            pltpu.VMEM((G, D), jnp.float32),
            pltpu.VMEM((128,), jnp.int32),
            pltpu.VMEM((C + LANE, LANE), jnp.int32),
            pltpu.SemaphoreType.DMA,
            pltpu.SemaphoreType.DMA,
        ],
    )(_sc_agg_body)


def _scalers(sb, qb, xb, nb, db):
    degc = jnp.maximum(db, 1.0)
    mean = sb / degc
    var = jnp.maximum(qb / degc - mean * mean, 0.0)
    std = jnp.sqrt(var + 1e-5)
    has = db > 0.0
    mxm = jnp.where(has, xb, 0.0)
    mnm = jnp.where(has, nb, 0.0)
    logd = jnp.log(db + 1.0)
    amp = logd / AVG_D_LOG
    att = jnp.where(has, AVG_D_LOG / jnp.maximum(logd, 1e-6), 1.0)
    agg = jnp.concatenate([mean, mxm, mnm, std], axis=1)
    ampt = jnp.concatenate([amp] * 4, axis=1)
    attt = jnp.concatenate([att] * 4, axis=1)
    return agg, ampt, attt


def _mm_body(s_ref, q_ref, x_ref, n_ref, d_ref, w_ref, b_ref,
             out_ref, cs_ref, cq_ref):
    i = pl.program_id(0)
    agg, ampt, attt = _scalers(s_ref[...], q_ref[...], x_ref[...], n_ref[...],
                               d_ref[...])
    w = w_ref[...]
    out = jnp.dot(agg, w[0:4 * D], preferred_element_type=jnp.float32)
    out += jnp.dot(agg * ampt, w[4 * D:8 * D], preferred_element_type=jnp.float32)
    out += jnp.dot(agg * attt, w[8 * D:12 * D], preferred_element_type=jnp.float32)
    out += b_ref[...]
    out_ref[...] = out

    @pl.when(i == 0)
    def _():
        cs_ref[...] = jnp.zeros_like(cs_ref)
        cq_ref[...] = jnp.zeros_like(cq_ref)

    rid = lax.broadcasted_iota(jnp.int32, (R, D), 0) + i * R
    om = jnp.where(rid < N, out, 0.0)
    cs_ref[...] += jnp.sum(om, axis=0, keepdims=True)
    cq_ref[...] += jnp.sum(om * om, axis=0, keepdims=True)


def _mm_last_body(s_ref, q_ref, x_ref, n_ref, d_ref, w_ref, b_ref, h_ref,
                  out_ref):
    agg, ampt, attt = _scalers(s_ref[...], q_ref[...], x_ref[...], n_ref[...],
                               d_ref[...])
    w = w_ref[...]
    out = jnp.dot(agg, w[0:4 * D], preferred_element_type=jnp.float32)
    out += jnp.dot(agg * ampt, w[4 * D:8 * D], preferred_element_type=jnp.float32)
    out += jnp.dot(agg * attt, w[8 * D:12 * D], preferred_element_type=jnp.float32)
    out_ref[...] = out + b_ref[...] + h_ref[...]


def _bn_body(out_ref, h_ref, cs_ref, cq_ref, g_ref, b_ref, new_ref):
    mu = cs_ref[...] / N
    var = cq_ref[...] / N - mu * mu
    inv = lax.rsqrt(var + 1e-5)
    y = (out_ref[...] - mu) * inv * g_ref[...] + b_ref[...]
    new_ref[...] = h_ref[...] + jnp.maximum(y, 0.0)


_row_spec = pl.BlockSpec((R, D), lambda i: (i, 0))
_full_w = pl.BlockSpec((12 * D, D), lambda i: (0, 0))
_vec_spec = pl.BlockSpec((1, D), lambda i: (0, 0))

_mm_call = pl.pallas_call(
    _mm_body,
    grid=(NRB,),
    in_specs=[_row_spec] * 5 + [_full_w, _vec_spec],
    out_specs=[_row_spec, _vec_spec, _vec_spec],
    out_shape=[
        jax.ShapeDtypeStruct((NP, D), jnp.float32),
        jax.ShapeDtypeStruct((1, D), jnp.float32),
        jax.ShapeDtypeStruct((1, D), jnp.float32),
    ],
)

_mm_last_call = pl.pallas_call(
    _mm_last_body,
    grid=(NRB,),
    in_specs=[_row_spec] * 5 + [_full_w, _vec_spec, _row_spec],
    out_specs=_row_spec,
    out_shape=jax.ShapeDtypeStruct((NP, D), jnp.float32),
)

_bn_call = pl.pallas_call(
    _bn_body,
    grid=(NRB,),
    in_specs=[_row_spec, _row_spec, _vec_spec, _vec_spec, _vec_spec, _vec_spec],
    out_specs=_row_spec,
    out_shape=jax.ShapeDtypeStruct((NP, D), jnp.float32),
)


def kernel(h, e, W0, b0, W1, b1, W2, b2, W3, b3,
           gamma0, beta0, gamma1, beta1, gamma2, beta2, edge_index):
    del e
    src = edge_index[0]
    dst = edge_index[1]
    Ws = [W0, W1, W2, W3]
    bs = [b.reshape(1, D) for b in (b0, b1, b2, b3)]
    gammas = [g.reshape(1, D) for g in (gamma0, gamma1, gamma2)]
    betas = [b.reshape(1, D) for b in (beta0, beta1, beta2)]

    _, _, bsrc, counts, dg, deg16 = _bucket_call()(src, dst)
    x = jnp.pad(h, ((0, NP - N), (0, 0)))
    for i in range(4):
        s, q, mx, mn = _sc_agg_call()(x, bsrc, deg16, counts)
        if i < 3:
            out, cs, cq = _mm_call(s, q, mx, mn, dg, Ws[i], bs[i])
            x = _bn_call(out, x, cs, cq, gammas[i], betas[i])
        else:
            x = _mm_last_call(s, q, mx, mn, dg, Ws[i], bs[i], x)
    return x[:N]
